# baseline (speedup 1.0000x reference)
import jax
import jax.numpy as jnp
from jax import lax
from jax.experimental import pallas as pl
from jax.experimental.pallas import tpu as pltpu
from jax.experimental.pallas import tpu_sc as plsc

VOCAB = 1000000
CHAR_VOCAB = 20000
DIM = 64
B = 16384
NEG = 5
MAXWL = 8
WCOL = 2 + NEG + NEG
CCOL = MAXWL + 1

NC = 2
NS = 16
NW = NC * NS
ROWS_PER_W = B // NW
CHUNK = 32
NCHUNK = ROWS_PER_W // CHUNK
LANES = 16
NGRP = ROWS_PER_W // LANES
KV = DIM // LANES
NSEC = 2 + NEG + MAXWL
WCOLS = (1, 0, 2, 3, 4, 5, 6)


def _sc_body(word_hbm, char_hbm,
             emb0, emb1, emb0c,
             pos_out, neg_out,
             word_v, char_v, idx_all, num_all, rows_r,
             pos_all, negb_all, sem0, sem1):
    wid = lax.axis_index("s") * NC + lax.axis_index("c")
    wbase = wid * ROWS_PER_W

    pltpu.sync_copy(word_hbm.at[pl.ds(wbase, ROWS_PER_W)], word_v)
    pltpu.sync_copy(char_hbm.at[pl.ds(wbase, ROWS_PER_W)], char_v)

    def extract_body(g, _):
        rvec = g * LANES + lax.iota(jnp.int32, LANES)
        c = g // (CHUNK // LANES)
        off = (g % (CHUNK // LANES)) * LANES
        for s in range(NSEC):
            if s < len(WCOLS):
                src, col = word_v, WCOLS[s]
            else:
                src, col = char_v, s - len(WCOLS)
            colvec = jnp.full((LANES,), col, jnp.int32)
            idx_all[s, c, pl.ds(off, LANES)] = plsc.load_gather(
                src, [rvec, colvec])
        nv = plsc.load_gather(char_v,
                              [rvec, jnp.full((LANES,), MAXWL, jnp.int32)])
        num_all[c, pl.ds(off, LANES)] = nv.astype(jnp.float32)
        return 0

    lax.fori_loop(0, NGRP, extract_body, 0)

    tables = [emb0, emb1] + [emb1] * NEG + [emb0c] * MAXWL
    sems = (sem0, sem1)

    def fire(c, b):
        for s in range(NSEC):
            pltpu.async_copy(tables[s].at[idx_all.at[s, c]],
                             rows_r.at[b, s], sems[b])

    def drain(b):
        for s in range(NSEC):
            pltpu.make_async_copy(tables[s].at[idx_all.at[s, 0]],
                                  rows_r.at[b, s], sems[b]).wait()

    def compute(c, b):
        def group_body(g, carry):
            invv = 0.5 / num_all[c, pl.ds(g * LANES, LANES)]
            lane_iota = lax.iota(jnp.int32, LANES)
            posvec = jnp.zeros((LANES,), jnp.float32)
            negvecs = [jnp.zeros((LANES,), jnp.float32) for _ in range(NEG)]
            for l in range(LANES):
                r = g * LANES + l
                cs = [rows_r[b, 7, r, pl.ds(16 * k, 16)] for k in range(KV)]
                for j in range(1, MAXWL):
                    cs = [cs[k] + rows_r[b, 7 + j, r, pl.ds(16 * k, 16)]
                          for k in range(KV)]
                inv = invv[l]
                avg = [rows_r[b, 0, r, pl.ds(16 * k, 16)] * 0.5 + cs[k] * inv
                       for k in range(KV)]
                acc = avg[0] * rows_r[b, 1, r, pl.ds(0, 16)]
                for k in range(1, KV):
                    acc = acc + avg[k] * rows_r[b, 1, r, pl.ds(16 * k, 16)]
                sel = lane_iota == l
                posvec = jnp.where(sel, jnp.sum(acc), posvec)
                for j in range(NEG):
                    accn = avg[0] * rows_r[b, 2 + j, r, pl.ds(0, 16)]
                    for k in range(1, KV):
                        accn = accn + avg[k] * rows_r[b, 2 + j, r,
                                                      pl.ds(16 * k, 16)]
                    negvecs[j] = jnp.where(sel, jnp.sum(accn), negvecs[j])
            obase = c * CHUNK + g * LANES
            pos_all[pl.ds(obase, LANES)] = posvec
            for j in range(NEG):
                sidx = (obase + lane_iota) * NEG + j
                plsc.store_scatter(negb_all, [sidx], negvecs[j])
            return carry

        lax.fori_loop(0, CHUNK // LANES, group_body, 0)

    fire(0, 0)

    def body2(cc, _):
        c0 = cc * 2
        fire(c0 + 1, 1)
        drain(0)
        compute(c0, 0)

        @pl.when(c0 + 2 < NCHUNK)
        def _():
            fire(c0 + 2, 0)

        drain(1)
        compute(c0 + 1, 1)
        return 0

    lax.fori_loop(0, NCHUNK // 2, body2, 0)

    pltpu.sync_copy(pos_all, pos_out.at[pl.ds(wbase, ROWS_PER_W)])
    pltpu.sync_copy(negb_all,
                    neg_out.at[pl.ds(wbase * NEG, ROWS_PER_W * NEG)])


_sc_dots = pl.kernel(
    _sc_body,
    out_type=(
        jax.ShapeDtypeStruct((B,), jnp.float32),
        jax.ShapeDtypeStruct((B * NEG,), jnp.float32),
    ),
    mesh=plsc.VectorSubcoreMesh(core_axis_name="c", subcore_axis_name="s"),
    compiler_params=pltpu.CompilerParams(needs_layout_passes=False,
                                         use_tc_tiling_on_sc=False),
    scratch_types=[
        pltpu.VMEM((ROWS_PER_W, WCOL), jnp.int32),
        pltpu.VMEM((ROWS_PER_W, CCOL), jnp.int32),
        pltpu.VMEM((NSEC, NCHUNK, CHUNK), jnp.int32),
        pltpu.VMEM((NCHUNK, CHUNK), jnp.float32),
        pltpu.VMEM((2, NSEC, CHUNK, DIM), jnp.float32),
        pltpu.VMEM((ROWS_PER_W,), jnp.float32),
        pltpu.VMEM((ROWS_PER_W * NEG,), jnp.float32),
        pltpu.SemaphoreType.DMA,
        pltpu.SemaphoreType.DMA,
    ],
)


def _loss_body(pos_ref, neg_ref, mask_ref, out_ref):
    p = jnp.clip(pos_ref[...], -10.0, 10.0)
    pos_loss = jnp.sum(jnp.log1p(jnp.exp(-p)))
    z = jnp.clip(-neg_ref[...], -10.0, 10.0)
    neg_loss = jnp.sum(jnp.log1p(jnp.exp(-z)) * mask_ref[...])
    out_ref[0, 0] = pos_loss + neg_loss


def _tc_loss(pos2, neg2, mask2):
    return pl.pallas_call(
        _loss_body,
        out_shape=jax.ShapeDtypeStruct((1, 1), jnp.float32),
        out_specs=pl.BlockSpec(memory_space=pltpu.SMEM),
    )(pos2, neg2, mask2)


@jax.jit
def kernel(word_data, char_data, emb0, emb1, emb0_char):
    pos_ips, neg_ips = _sc_dots(word_data, char_data, emb0, emb1, emb0_char)

    mask2 = word_data[:, 2 + NEG:].astype(jnp.float32).reshape(
        B * NEG // 128, 128)
    loss = _tc_loss(pos_ips.reshape(B // 128, 128),
                    neg_ips.reshape(B * NEG // 128, 128),
                    mask2)
    return loss[0, 0]


# speedup vs baseline: 1.0596x; 1.0596x over previous
import jax
import jax.numpy as jnp
from jax import lax
from jax.experimental import pallas as pl
from jax.experimental.pallas import tpu as pltpu
from jax.experimental.pallas import tpu_sc as plsc

VOCAB = 1000000
CHAR_VOCAB = 20000
DIM = 64
B = 16384
NEG = 5
MAXWL = 8
WCOL = 2 + NEG + NEG
CCOL = MAXWL + 1

NC = 2
NS = 16
NW = NC * NS
ROWS_PER_W = B // NW
CHUNK = 64
NCHUNK = ROWS_PER_W // CHUNK
LANES = 16
NGRP = ROWS_PER_W // LANES
GPC = CHUNK // LANES
KV = DIM // LANES
NSEC = 2 + NEG + MAXWL
NBUF = 2 + NEG + 1
WCOLS = (1, 0, 2, 3, 4, 5, 6)


def _sc_body(word_hbm, char_hbm,
             emb0, emb1, emb0c,
             pos_out, neg_out,
             word_v, char_v, idx_all, num_all, rows_r,
             pos_all, negb_all, sem0, sem1):
    wid = lax.axis_index("s") * NC + lax.axis_index("c")
    wbase = wid * ROWS_PER_W

    pltpu.sync_copy(word_hbm.at[pl.ds(wbase, ROWS_PER_W)], word_v)
    pltpu.sync_copy(char_hbm.at[pl.ds(wbase, ROWS_PER_W)], char_v)

    def extract_body(g, _):
        rvec = g * LANES + lax.iota(jnp.int32, LANES)
        c = g // GPC
        off = (g % GPC) * LANES
        for s in range(NSEC):
            if s < len(WCOLS):
                src, col = word_v, WCOLS[s]
            else:
                src, col = char_v, s - len(WCOLS)
            colvec = jnp.full((LANES,), col, jnp.int32)
            idx_all[s, c, pl.ds(off, LANES)] = plsc.load_gather(
                src, [rvec, colvec])
        nv = plsc.load_gather(char_v,
                              [rvec, jnp.full((LANES,), MAXWL, jnp.int32)])
        num_all[c, pl.ds(off, LANES)] = nv.astype(jnp.float32)
        return 0

    lax.fori_loop(0, NGRP, extract_body, 0)

    def zero_body(r, _):
        for b in range(2):
            for k in range(KV):
                rows_r[b, 7, r, pl.ds(16 * k, 16)] = jnp.zeros((LANES,),
                                                               jnp.float32)
        return 0

    lax.fori_loop(0, CHUNK, zero_body, 0)

    tables = [emb0, emb1] + [emb1] * NEG
    sems = (sem0, sem1)

    def fire(c, b):
        for s in range(7):
            pltpu.async_copy(tables[s].at[idx_all.at[s, c]],
                             rows_r.at[b, s], sems[b])
        for j in range(MAXWL):
            pltpu.async_copy(emb0c.at[idx_all.at[7 + j, c]],
                             rows_r.at[b, 7], sems[b], add=True)

    def drain(b):
        for s in range(7):
            pltpu.make_async_copy(tables[s].at[idx_all.at[s, 0]],
                                  rows_r.at[b, s], sems[b]).wait()
        for j in range(MAXWL):
            pltpu.make_async_copy(emb0c.at[idx_all.at[7, 0]],
                                  rows_r.at[b, 7], sems[b]).wait()

    def compute(c, b):
        def group_body(g, carry):
            invv = 0.5 / num_all[c, pl.ds(g * LANES, LANES)]
            lane_iota = lax.iota(jnp.int32, LANES)
            posvec = jnp.zeros((LANES,), jnp.float32)
            negvecs = [jnp.zeros((LANES,), jnp.float32) for _ in range(NEG)]
            zero16 = jnp.zeros((LANES,), jnp.float32)
            for l in range(LANES):
                r = g * LANES + l
                inv = invv[l]
                avg = []
                for k in range(KV):
                    csk = rows_r[b, 7, r, pl.ds(16 * k, 16)]
                    avg.append(rows_r[b, 0, r, pl.ds(16 * k, 16)] * 0.5
                               + csk * inv)
                    rows_r[b, 7, r, pl.ds(16 * k, 16)] = zero16
                acc = avg[0] * rows_r[b, 1, r, pl.ds(0, 16)]
                for k in range(1, KV):
                    acc = acc + avg[k] * rows_r[b, 1, r, pl.ds(16 * k, 16)]
                sel = lane_iota == l
                posvec = jnp.where(sel, jnp.sum(acc), posvec)
                for j in range(NEG):
                    accn = avg[0] * rows_r[b, 2 + j, r, pl.ds(0, 16)]
                    for k in range(1, KV):
                        accn = accn + avg[k] * rows_r[b, 2 + j, r,
                                                      pl.ds(16 * k, 16)]
                    negvecs[j] = jnp.where(sel, jnp.sum(accn), negvecs[j])
            obase = c * CHUNK + g * LANES
            pos_all[pl.ds(obase, LANES)] = posvec
            for j in range(NEG):
                sidx = (obase + lane_iota) * NEG + j
                plsc.store_scatter(negb_all, [sidx], negvecs[j])
            return carry

        lax.fori_loop(0, GPC, group_body, 0)

    fire(0, 0)

    def body2(cc, _):
        c0 = cc * 2
        fire(c0 + 1, 1)
        drain(0)
        compute(c0, 0)

        @pl.when(c0 + 2 < NCHUNK)
        def _():
            fire(c0 + 2, 0)

        drain(1)
        compute(c0 + 1, 1)
        return 0

    lax.fori_loop(0, NCHUNK // 2, body2, 0)

    pltpu.sync_copy(pos_all, pos_out.at[pl.ds(wbase, ROWS_PER_W)])
    pltpu.sync_copy(negb_all,
                    neg_out.at[pl.ds(wbase * NEG, ROWS_PER_W * NEG)])


_sc_dots = pl.kernel(
    _sc_body,
    out_type=(
        jax.ShapeDtypeStruct((B,), jnp.float32),
        jax.ShapeDtypeStruct((B * NEG,), jnp.float32),
    ),
    mesh=plsc.VectorSubcoreMesh(core_axis_name="c", subcore_axis_name="s"),
    compiler_params=pltpu.CompilerParams(needs_layout_passes=False,
                                         use_tc_tiling_on_sc=False),
    scratch_types=[
        pltpu.VMEM((ROWS_PER_W, WCOL), jnp.int32),
        pltpu.VMEM((ROWS_PER_W, CCOL), jnp.int32),
        pltpu.VMEM((NSEC, NCHUNK, CHUNK), jnp.int32),
        pltpu.VMEM((NCHUNK, CHUNK), jnp.float32),
        pltpu.VMEM((2, NBUF, CHUNK, DIM), jnp.float32),
        pltpu.VMEM((ROWS_PER_W,), jnp.float32),
        pltpu.VMEM((ROWS_PER_W * NEG,), jnp.float32),
        pltpu.SemaphoreType.DMA,
        pltpu.SemaphoreType.DMA,
    ],
)


def _loss_body(pos_ref, neg_ref, mask_ref, out_ref):
    p = jnp.clip(pos_ref[...], -10.0, 10.0)
    pos_loss = jnp.sum(jnp.log1p(jnp.exp(-p)))
    z = jnp.clip(-neg_ref[...], -10.0, 10.0)
    neg_loss = jnp.sum(jnp.log1p(jnp.exp(-z)) * mask_ref[...])
    out_ref[0, 0] = pos_loss + neg_loss


def _tc_loss(pos2, neg2, mask2):
    return pl.pallas_call(
        _loss_body,
        out_shape=jax.ShapeDtypeStruct((1, 1), jnp.float32),
        out_specs=pl.BlockSpec(memory_space=pltpu.SMEM),
    )(pos2, neg2, mask2)


@jax.jit
def kernel(word_data, char_data, emb0, emb1, emb0_char):
    pos_ips, neg_ips = _sc_dots(word_data, char_data, emb0, emb1, emb0_char)

    mask2 = word_data[:, 2 + NEG:].astype(jnp.float32).reshape(
        B * NEG // 128, 128)
    loss = _tc_loss(pos_ips.reshape(B // 128, 128),
                    neg_ips.reshape(B * NEG // 128, 128),
                    mask2)
    return loss[0, 0]
